# baseline (speedup 1.0000x reference)
import functools
import math

import jax
import jax.numpy as jnp
from jax import lax
from jax.experimental import pallas as pl
from jax.experimental.pallas import tpu as pltpu
from jax.experimental.pallas import tpu_sc as plsc

N = 2048
D = 1024
F = 2048
E = 8
TB = 256
NT = N // TB
NTILES = N * 2 // TB + E
PAD = NTILES * TB
PB = 512
NW = 32
TPW = N // NW
CH = 32


def _route_pos_body(h_ref, wg_ref, bg_ref, pos_ref, wc_ref, tmap_ref):
    logits = lax.dot_general(
        h_ref[...], wg_ref[...], (((1,), (1,)), ((), ())),
        preferred_element_type=jnp.float32) + bg_ref[...]
    ids = lax.broadcasted_iota(jnp.int32, (N, E), 1)
    m1 = jnp.max(logits, axis=1, keepdims=True)
    i1 = jnp.min(jnp.where(logits == m1, ids, E), axis=1, keepdims=True)
    l2 = jnp.where(ids == i1, -jnp.inf, logits)
    m2 = jnp.max(l2, axis=1, keepdims=True)
    i2 = jnp.min(jnp.where(l2 == m2, ids, E), axis=1, keepdims=True)
    w1 = 1.0 / (1.0 + jnp.exp(m2 - m1))
    wc_ref[...] = jnp.concatenate([w1, 1.0 - w1], axis=1)

    erow = lax.broadcasted_iota(jnp.int32, (1, E), 1)
    rr = lax.broadcasted_iota(jnp.int32, (PB, PB), 0)
    cc = lax.broadcasted_iota(jnp.int32, (PB, PB), 1)
    lmat = (cc < rr).astype(jnp.float32)
    nb = 2 * N // PB
    masks, runs, run = [], [], jnp.zeros((1, E), jnp.float32)
    for b in range(nb):
        col = (i1 if b < nb // 2 else i2)
        seg = col[(b % (nb // 2)) * PB:(b % (nb // 2)) * PB + PB, :]
        m = (seg == erow).astype(jnp.float32)
        masks.append(m)
        runs.append(run)
        run = run + jnp.sum(m, axis=0, keepdims=True)
    totals = run
    m_all = jnp.concatenate(masks, axis=1)
    p_all = lax.dot_general(lmat, m_all, (((1,), (0,)), ((), ())),
                            preferred_element_type=jnp.float32)
    prefs = [p_all[:, b * E:(b + 1) * E] for b in range(nb)]
    pcnt = jnp.floor((totals + (TB - 1.0)) * (1.0 / TB)) * TB
    r8 = lax.broadcasted_iota(jnp.int32, (E, E), 0)
    c8 = lax.broadcasted_iota(jnp.int32, (E, E), 1)
    po = lax.dot_general(pcnt, (r8 < c8).astype(jnp.float32),
                         (((1,), (0,)), ((), ())),
                         preferred_element_type=jnp.float32)
    cum_end = lax.dot_general(pcnt, (r8 <= c8).astype(jnp.float32),
                              (((1,), (0,)), ((), ())),
                              preferred_element_type=jnp.float32)
    tile_iota = lax.broadcasted_iota(
        jnp.int32, (1, NTILES + 8), 1).astype(jnp.float32)
    te_acc = jnp.zeros((1, NTILES + 8), jnp.float32)
    for ex in range(E):
        te_acc = te_acc + (
            tile_iota >= cum_end[:, ex:ex + 1] * (1.0 / TB)
        ).astype(jnp.float32)
    tmap_ref[...] = te_acc.astype(jnp.int32)

    for b in range(nb):
        slot = jnp.sum(masks[b] * (po + runs[b] + prefs[b]),
                       axis=1, keepdims=True)
        k = 0 if b < nb // 2 else 1
        r0 = (b % (nb // 2)) * PB
        pos_ref[r0:r0 + PB, k:k + 1] = slot.astype(jnp.int32)


def _dispatch_body(h_hbm, post_hbm, x_hbm, idx0_v, idx1_v, rows_v,
                   sem0, sem1):
    wid = lax.axis_index("s") * 2 + lax.axis_index("c")
    base = wid * TPW
    pltpu.sync_copy(h_hbm.at[pl.ds(base, TPW)], rows_v)
    pltpu.sync_copy(post_hbm.at[0, pl.ds(base, TPW)], idx0_v)
    pltpu.sync_copy(post_hbm.at[1, pl.ds(base, TPW)], idx1_v)
    cp0 = pltpu.async_copy(rows_v, x_hbm.at[idx0_v], sem0)
    cp1 = pltpu.async_copy(rows_v, x_hbm.at[idx1_v], sem1)
    cp0.wait()
    cp1.wait()


def _gelu(x):
    return 0.5 * x * (1.0 + lax.erf(x * (1.0 / math.sqrt(2.0))))


def _ffn_body(tm_ref, x_ref, w1a_ref, w1b_ref, b1_ref, w2a_ref, w2b_ref,
              b2_ref, y_ref):
    j = pl.program_id(0)

    @pl.when(tm_ref[j] < E)
    def _compute():
        h1a = lax.dot_general(
            x_ref[...], w1a_ref[0], (((1,), (1,)), ((), ())),
            preferred_element_type=jnp.float32)
        h1b = lax.dot_general(
            x_ref[...], w1b_ref[0], (((1,), (1,)), ((), ())),
            preferred_element_type=jnp.float32)
        h1 = jnp.concatenate([h1a, h1b], axis=1) + b1_ref[0]
        act = _gelu(h1)
        y_ref[...] = (
            lax.dot_general(
                act[:, :F // 2], w2a_ref[0], (((1,), (1,)), ((), ())),
                preferred_element_type=jnp.float32)
            + lax.dot_general(
                act[:, F // 2:], w2b_ref[0], (((1,), (1,)), ((), ())),
                preferred_element_type=jnp.float32)
            + b2_ref[0])


def _collect_body(y_hbm, post_hbm, y0_hbm, y1_hbm,
                  idx0_v, idx1_v, b0_v, b1_v, sem0, sem1, semw):
    wid = lax.axis_index("s") * 2 + lax.axis_index("c")
    for c in range(TPW // CH):
        cb = wid * TPW + c * CH
        pltpu.sync_copy(post_hbm.at[0, pl.ds(cb, CH)], idx0_v)
        pltpu.sync_copy(post_hbm.at[1, pl.ds(cb, CH)], idx1_v)
        g0 = pltpu.async_copy(y_hbm.at[idx0_v], b0_v, sem0)
        g1 = pltpu.async_copy(y_hbm.at[idx1_v], b1_v, sem1)
        g0.wait()
        w0 = pltpu.async_copy(b0_v, y0_hbm.at[pl.ds(cb, CH)], semw)
        g1.wait()
        w1 = pltpu.async_copy(b1_v, y1_hbm.at[pl.ds(cb, CH)], semw)
        w0.wait()
        w1.wait()


def _proj_body(y0_ref, y1_ref, wc_ref, wp_ref, bp_ref, out_ref):
    wc = wc_ref[...]
    x = wc[:, 0:1] * y0_ref[...] + wc[:, 1:2] * y1_ref[...]
    out_ref[...] = lax.dot_general(
        x, wp_ref[...], (((1,), (1,)), ((), ())),
        preferred_element_type=jnp.float32) + bp_ref[...]


@jax.jit
def kernel(h_c, Wg, bg, W1, b1, W2, b2, Wp, bp):
    sc_mesh = plsc.VectorSubcoreMesh(core_axis_name="c", subcore_axis_name="s")

    pos, wc, tmap = pl.pallas_call(
        _route_pos_body,
        out_shape=(jax.ShapeDtypeStruct((N, 2), jnp.int32),
                   jax.ShapeDtypeStruct((N, 2), jnp.float32),
                   jax.ShapeDtypeStruct((1, NTILES + 8), jnp.int32)),
    )(h_c, Wg, bg.reshape(1, E))

    post = pos.T
    tmap1d = tmap.reshape(NTILES + 8)[:NTILES]

    dispatch = functools.partial(
        pl.kernel,
        out_type=jax.ShapeDtypeStruct((PAD, D), jnp.float32),
        mesh=sc_mesh,
        scratch_types=[
            pltpu.VMEM((TPW,), jnp.int32),
            pltpu.VMEM((TPW,), jnp.int32),
            pltpu.VMEM((TPW, D), jnp.float32),
            pltpu.SemaphoreType.DMA,
            pltpu.SemaphoreType.DMA,
        ],
    )(_dispatch_body)
    x_pad = dispatch(h_c, post)

    y_pad = pl.pallas_call(
        _ffn_body,
        grid_spec=pltpu.PrefetchScalarGridSpec(
            num_scalar_prefetch=1,
            grid=(NTILES,),
            in_specs=[
                pl.BlockSpec((TB, D), lambda j, tm: (j, 0)),
                pl.BlockSpec((1, F // 2, D),
                             lambda j, tm: (jnp.minimum(tm[j], E - 1), 0, 0)),
                pl.BlockSpec((1, F // 2, D),
                             lambda j, tm: (jnp.minimum(tm[j], E - 1), 1, 0)),
                pl.BlockSpec((1, 1, F),
                             lambda j, tm: (jnp.minimum(tm[j], E - 1), 0, 0)),
                pl.BlockSpec((1, D, F // 2),
                             lambda j, tm: (jnp.minimum(tm[j], E - 1), 0, 0)),
                pl.BlockSpec((1, D, F // 2),
                             lambda j, tm: (jnp.minimum(tm[j], E - 1), 0, 1)),
                pl.BlockSpec((1, 1, D),
                             lambda j, tm: (jnp.minimum(tm[j], E - 1), 0, 0)),
            ],
            out_specs=pl.BlockSpec((TB, D), lambda j, tm: (j, 0)),
        ),
        out_shape=jax.ShapeDtypeStruct((PAD, D), jnp.float32),
    )(tmap1d, x_pad, W1, W1, b1.reshape(E, 1, F), W2, W2,
      b2.reshape(E, 1, D))

    collect = functools.partial(
        pl.kernel,
        out_type=(jax.ShapeDtypeStruct((N, D), jnp.float32),
                  jax.ShapeDtypeStruct((N, D), jnp.float32)),
        mesh=sc_mesh,
        scratch_types=[
            pltpu.VMEM((CH,), jnp.int32),
            pltpu.VMEM((CH,), jnp.int32),
            pltpu.VMEM((CH, D), jnp.float32),
            pltpu.VMEM((CH, D), jnp.float32),
            pltpu.SemaphoreType.DMA,
            pltpu.SemaphoreType.DMA,
            pltpu.SemaphoreType.DMA,
        ],
    )(_collect_body)
    y0, y1 = collect(y_pad, post)

    out = pl.pallas_call(
        _proj_body,
        grid=(NT,),
        in_specs=[
            pl.BlockSpec((TB, D), lambda nt: (nt, 0)),
            pl.BlockSpec((TB, D), lambda nt: (nt, 0)),
            pl.BlockSpec((TB, 2), lambda nt: (nt, 0)),
            pl.BlockSpec((D, D), lambda nt: (0, 0)),
            pl.BlockSpec((1, D), lambda nt: (0, 0)),
        ],
        out_specs=pl.BlockSpec((TB, D), lambda nt: (nt, 0)),
        out_shape=jax.ShapeDtypeStruct((N, D), jnp.float32),
    )(y0, y1, wc, Wp, bp.reshape(1, D))
    return out


# speedup vs baseline: 1.0038x; 1.0038x over previous
import functools
import math

import jax
import jax.numpy as jnp
from jax import lax
from jax.experimental import pallas as pl
from jax.experimental.pallas import tpu as pltpu
from jax.experimental.pallas import tpu_sc as plsc

N = 2048
D = 1024
F = 2048
E = 8
TB = 256
NT = N // TB
NTILES = N * 2 // TB + E
PAD = NTILES * TB
PB = 512
NW = 32
TPW = N // NW
CH = 32


def _route_pos_body(h_ref, wg_ref, bg_ref, pos_ref, wc_ref, tmap_ref):
    logits = lax.dot_general(
        h_ref[...], wg_ref[...], (((1,), (1,)), ((), ())),
        preferred_element_type=jnp.float32) + bg_ref[...]
    ids = lax.broadcasted_iota(jnp.int32, (N, E), 1)
    m1 = jnp.max(logits, axis=1, keepdims=True)
    i1 = jnp.min(jnp.where(logits == m1, ids, E), axis=1, keepdims=True)
    l2 = jnp.where(ids == i1, -jnp.inf, logits)
    m2 = jnp.max(l2, axis=1, keepdims=True)
    i2 = jnp.min(jnp.where(l2 == m2, ids, E), axis=1, keepdims=True)
    w1 = 1.0 / (1.0 + jnp.exp(m2 - m1))
    wc_ref[...] = jnp.concatenate([w1, 1.0 - w1], axis=1)

    erow = lax.broadcasted_iota(jnp.int32, (1, E), 1)
    rr = lax.broadcasted_iota(jnp.int32, (PB, PB), 0)
    cc = lax.broadcasted_iota(jnp.int32, (PB, PB), 1)
    lmat = (cc < rr).astype(jnp.float32)
    nb = 2 * N // PB
    masks, runs, run = [], [], jnp.zeros((1, E), jnp.float32)
    for b in range(nb):
        col = (i1 if b < nb // 2 else i2)
        seg = col[(b % (nb // 2)) * PB:(b % (nb // 2)) * PB + PB, :]
        m = (seg == erow).astype(jnp.float32)
        masks.append(m)
        runs.append(run)
        run = run + jnp.sum(m, axis=0, keepdims=True)
    totals = run
    m_all = jnp.concatenate(masks, axis=1)
    p_all = lax.dot_general(lmat, m_all, (((1,), (0,)), ((), ())),
                            preferred_element_type=jnp.float32)
    prefs = [p_all[:, b * E:(b + 1) * E] for b in range(nb)]
    pcnt = jnp.floor((totals + (TB - 1.0)) * (1.0 / TB)) * TB
    r8 = lax.broadcasted_iota(jnp.int32, (E, E), 0)
    c8 = lax.broadcasted_iota(jnp.int32, (E, E), 1)
    po = lax.dot_general(pcnt, (r8 < c8).astype(jnp.float32),
                         (((1,), (0,)), ((), ())),
                         preferred_element_type=jnp.float32)
    cum_end = lax.dot_general(pcnt, (r8 <= c8).astype(jnp.float32),
                              (((1,), (0,)), ((), ())),
                              preferred_element_type=jnp.float32)
    tile_iota = lax.broadcasted_iota(
        jnp.int32, (1, NTILES + 8), 1).astype(jnp.float32)
    te_acc = jnp.zeros((1, NTILES + 8), jnp.float32)
    for ex in range(E):
        te_acc = te_acc + (
            tile_iota >= cum_end[:, ex:ex + 1] * (1.0 / TB)
        ).astype(jnp.float32)
    tmap_ref[...] = te_acc.astype(jnp.int32)

    for b in range(nb):
        slot = jnp.sum(masks[b] * (po + runs[b] + prefs[b]),
                       axis=1, keepdims=True)
        k = 0 if b < nb // 2 else 1
        r0 = (b % (nb // 2)) * PB
        pos_ref[r0:r0 + PB, k:k + 1] = slot.astype(jnp.int32)


def _dispatch_body(h_hbm, post_hbm, x_hbm, idx0_v, idx1_v, rows_v,
                   sem0, sem1):
    wid = lax.axis_index("s") * 2 + lax.axis_index("c")
    base = wid * TPW
    pltpu.sync_copy(h_hbm.at[pl.ds(base, TPW)], rows_v)
    pltpu.sync_copy(post_hbm.at[0, pl.ds(base, TPW)], idx0_v)
    pltpu.sync_copy(post_hbm.at[1, pl.ds(base, TPW)], idx1_v)
    cp0 = pltpu.async_copy(rows_v, x_hbm.at[idx0_v], sem0)
    cp1 = pltpu.async_copy(rows_v, x_hbm.at[idx1_v], sem1)
    cp0.wait()
    cp1.wait()


def _gelu(x):
    return 0.5 * x * (1.0 + lax.erf(x * (1.0 / math.sqrt(2.0))))


def _ffn_body(tm_ref, x_ref, w1_ref, b1_ref, w2_ref, b2_ref, y_ref):
    j = pl.program_id(0)

    @pl.when(tm_ref[j] < E)
    def _compute():
        h1 = lax.dot_general(
            x_ref[...], w1_ref[0], (((1,), (1,)), ((), ())),
            preferred_element_type=jnp.float32) + b1_ref[0]
        act = _gelu(h1)
        y_ref[...] = lax.dot_general(
            act, w2_ref[0], (((1,), (1,)), ((), ())),
            preferred_element_type=jnp.float32) + b2_ref[0]


def _collect_body(y_hbm, post_hbm, y0_hbm, y1_hbm,
                  idx0_v, idx1_v, b0_v, b1_v, sem0, sem1, semw):
    wid = lax.axis_index("s") * 2 + lax.axis_index("c")
    for c in range(TPW // CH):
        cb = wid * TPW + c * CH
        pltpu.sync_copy(post_hbm.at[0, pl.ds(cb, CH)], idx0_v)
        pltpu.sync_copy(post_hbm.at[1, pl.ds(cb, CH)], idx1_v)
        g0 = pltpu.async_copy(y_hbm.at[idx0_v], b0_v, sem0)
        g1 = pltpu.async_copy(y_hbm.at[idx1_v], b1_v, sem1)
        g0.wait()
        w0 = pltpu.async_copy(b0_v, y0_hbm.at[pl.ds(cb, CH)], semw)
        g1.wait()
        w1 = pltpu.async_copy(b1_v, y1_hbm.at[pl.ds(cb, CH)], semw)
        w0.wait()
        w1.wait()


def _proj_body(y0_ref, y1_ref, wc_ref, wp_ref, bp_ref, out_ref):
    wc = wc_ref[...]
    x = wc[:, 0:1] * y0_ref[...] + wc[:, 1:2] * y1_ref[...]
    out_ref[...] = lax.dot_general(
        x, wp_ref[...], (((1,), (1,)), ((), ())),
        preferred_element_type=jnp.float32) + bp_ref[...]


@jax.jit
def kernel(h_c, Wg, bg, W1, b1, W2, b2, Wp, bp):
    sc_mesh = plsc.VectorSubcoreMesh(core_axis_name="c", subcore_axis_name="s")

    pos, wc, tmap = pl.pallas_call(
        _route_pos_body,
        out_shape=(jax.ShapeDtypeStruct((N, 2), jnp.int32),
                   jax.ShapeDtypeStruct((N, 2), jnp.float32),
                   jax.ShapeDtypeStruct((1, NTILES + 8), jnp.int32)),
    )(h_c, Wg, bg.reshape(1, E))

    post = pos.T
    tmap1d = tmap.reshape(NTILES + 8)[:NTILES]

    dispatch = functools.partial(
        pl.kernel,
        out_type=jax.ShapeDtypeStruct((PAD, D), jnp.float32),
        mesh=sc_mesh,
        scratch_types=[
            pltpu.VMEM((TPW,), jnp.int32),
            pltpu.VMEM((TPW,), jnp.int32),
            pltpu.VMEM((TPW, D), jnp.float32),
            pltpu.SemaphoreType.DMA,
            pltpu.SemaphoreType.DMA,
        ],
    )(_dispatch_body)
    x_pad = dispatch(h_c, post)

    y_pad = pl.pallas_call(
        _ffn_body,
        grid_spec=pltpu.PrefetchScalarGridSpec(
            num_scalar_prefetch=1,
            grid=(NTILES,),
            in_specs=[
                pl.BlockSpec((TB, D), lambda j, tm: (j, 0)),
                pl.BlockSpec((1, F, D),
                             lambda j, tm: (jnp.minimum(tm[j], E - 1), 0, 0)),
                pl.BlockSpec((1, 1, F),
                             lambda j, tm: (jnp.minimum(tm[j], E - 1), 0, 0)),
                pl.BlockSpec((1, D, F),
                             lambda j, tm: (jnp.minimum(tm[j], E - 1), 0, 0)),
                pl.BlockSpec((1, 1, D),
                             lambda j, tm: (jnp.minimum(tm[j], E - 1), 0, 0)),
            ],
            out_specs=pl.BlockSpec((TB, D), lambda j, tm: (j, 0)),
        ),
        out_shape=jax.ShapeDtypeStruct((PAD, D), jnp.float32),
    )(tmap1d, x_pad, W1, b1.reshape(E, 1, F), W2, b2.reshape(E, 1, D))

    collect = functools.partial(
        pl.kernel,
        out_type=(jax.ShapeDtypeStruct((N, D), jnp.float32),
                  jax.ShapeDtypeStruct((N, D), jnp.float32)),
        mesh=sc_mesh,
        scratch_types=[
            pltpu.VMEM((CH,), jnp.int32),
            pltpu.VMEM((CH,), jnp.int32),
            pltpu.VMEM((CH, D), jnp.float32),
            pltpu.VMEM((CH, D), jnp.float32),
            pltpu.SemaphoreType.DMA,
            pltpu.SemaphoreType.DMA,
            pltpu.SemaphoreType.DMA,
        ],
    )(_collect_body)
    y0, y1 = collect(y_pad, post)

    out = pl.pallas_call(
        _proj_body,
        grid=(NT,),
        in_specs=[
            pl.BlockSpec((TB, D), lambda nt: (nt, 0)),
            pl.BlockSpec((TB, D), lambda nt: (nt, 0)),
            pl.BlockSpec((TB, 2), lambda nt: (nt, 0)),
            pl.BlockSpec((D, D), lambda nt: (0, 0)),
            pl.BlockSpec((1, D), lambda nt: (0, 0)),
        ],
        out_specs=pl.BlockSpec((TB, D), lambda nt: (nt, 0)),
        out_shape=jax.ShapeDtypeStruct((N, D), jnp.float32),
    )(y0, y1, wc, Wp, bp.reshape(1, D))
    return out
